# Initial kernel scaffold; baseline (speedup 1.0000x reference)
import jax
import jax.numpy as jnp
from jax.experimental import pallas as pl


def kernel(boxes, scores, box_deltas):
    raise NotImplementedError("write your pallas kernel here")


# speedup vs baseline: 10.1212x; 10.1212x over previous
import functools

import jax
import jax.numpy as jnp
from jax import lax
from jax.experimental import pallas as pl
from jax.experimental.pallas import tpu as pltpu
from jax.experimental.pallas import tpu_sc as plsc

N = 20000
NPAD = 20480
SUBS = 16
LANES = 16
E = NPAD // SUBS
CHUNKS = E // LANES
KEEP = 100
IMG = 1024.0
NEG = -1e9
PAD_NEG = -3e9
BIG = 3.0e38


def _build_cand(rmax, rgid, lane, x1v, y1v, x2v, y2v, arv, base):
    m = jnp.max(rmax)
    mv = jnp.full((LANES,), m, jnp.float32)
    g = jnp.min(jnp.where(rmax == mv, rgid, BIG))
    gv = jnp.full((LANES,), g, jnp.float32)
    lidx = g.astype(jnp.int32) - base
    iv = jnp.full((LANES,), lidx, jnp.int32)
    x1w = plsc.load_gather(x1v, [iv])
    y1w = plsc.load_gather(y1v, [iv])
    x2w = plsc.load_gather(x2v, [iv])
    y2w = plsc.load_gather(y2v, [iv])
    arw = plsc.load_gather(arv, [iv])
    rec = jnp.where(
        lane == 0, mv,
        jnp.where(lane == 1, x1w,
                  jnp.where(lane == 2, y1w,
                            jnp.where(lane == 3, x2w,
                                      jnp.where(lane == 4, y2w,
                                                jnp.where(lane == 5, arw, gv))))))
    return rec


def _nms_body(bx1_h, by1_h, bx2_h, by2_h, dx_h, dy_h, dw_h, dh_h, sc_h,
              out_h,
              x1v, y1v, x2v, y2v, dxv, dyv, dwv, dhv, arv, wkv,
              candv, allc, outv, shared):
    sub = lax.axis_index("s")
    core = lax.axis_index("c")
    base = sub * E
    lane = lax.broadcasted_iota(jnp.int32, (LANES,), 0)
    rmax0 = jnp.full((LANES,), -BIG, jnp.float32)
    rgid0 = jnp.zeros((LANES,), jnp.float32)

    pltpu.sync_copy(bx1_h.at[pl.ds(base, E)], x1v)
    pltpu.sync_copy(by1_h.at[pl.ds(base, E)], y1v)
    pltpu.sync_copy(bx2_h.at[pl.ds(base, E)], x2v)
    pltpu.sync_copy(by2_h.at[pl.ds(base, E)], y2v)
    pltpu.sync_copy(dx_h.at[pl.ds(base, E)], dxv)
    pltpu.sync_copy(dy_h.at[pl.ds(base, E)], dyv)
    pltpu.sync_copy(dw_h.at[pl.ds(base, E)], dwv)
    pltpu.sync_copy(dh_h.at[pl.ds(base, E)], dhv)
    pltpu.sync_copy(sc_h.at[pl.ds(base, E)], wkv)

    def decode_chunk(j, carry):
        rmax, rgid = carry
        sl = pl.ds(j * LANES, LANES)
        x1 = x1v[sl]
        y1 = y1v[sl]
        x2 = x2v[sl]
        y2 = y2v[sl]
        w = x2 - x1 + 1.0
        h = y2 - y1 + 1.0
        cx = x1 + 0.5 * w
        cy = y1 + 0.5 * h
        pcx = dxv[sl] * w + cx
        pcy = dyv[sl] * h + cy
        pw = jnp.exp(dwv[sl]) * w
        ph = jnp.exp(dhv[sl]) * h
        nx1 = jnp.clip(pcx - 0.5 * pw, 0.0, IMG)
        ny1 = jnp.clip(pcy - 0.5 * ph, 0.0, IMG)
        nx2 = jnp.clip(pcx + 0.5 * pw, 0.0, IMG)
        ny2 = jnp.clip(pcy + 0.5 * ph, 0.0, IMG)
        ar = jnp.maximum(nx2 - nx1, 0.0) * jnp.maximum(ny2 - ny1, 0.0)
        x1v[sl] = nx1
        y1v[sl] = ny1
        x2v[sl] = nx2
        y2v[sl] = ny2
        arv[sl] = ar
        wk = wkv[sl]
        gidf = (base + j * LANES + lane).astype(jnp.float32)
        better = wk > rmax
        return (jnp.where(better, wk, rmax), jnp.where(better, gidf, rgid))

    rmax, rgid = lax.fori_loop(0, CHUNKS, decode_chunk, (rmax0, rgid0))
    rec0 = _build_cand(rmax, rgid, lane, x1v, y1v, x2v, y2v, arv, base)

    def step(t, rec):
        candv[...] = rec
        pltpu.sync_copy(candv, shared.at[pl.ds(sub * LANES, LANES)])
        plsc.subcore_barrier()
        pltpu.sync_copy(shared, allc)
        vals = plsc.load_gather(allc, [lane * LANES])
        gids = plsc.load_gather(allc, [lane * LANES + 6])
        m = jnp.max(vals)
        tie = vals == jnp.full((LANES,), m, jnp.float32)
        wg = jnp.min(jnp.where(tie, gids, BIG))
        rowc = jnp.where(tie & (gids == jnp.full((LANES,), wg, jnp.float32)),
                         lane, 9999)
        rb = jnp.min(rowc) * LANES
        win = plsc.load_gather(allc, [jnp.full((LANES,), rb, jnp.int32) + lane])

        @pl.when((sub == 0) & (core == 0))
        def _():
            outv[pl.ds(t * LANES, LANES)] = win

        bx1 = plsc.load_gather(allc, [jnp.full((LANES,), rb + 1, jnp.int32)])
        by1 = plsc.load_gather(allc, [jnp.full((LANES,), rb + 2, jnp.int32)])
        bx2 = plsc.load_gather(allc, [jnp.full((LANES,), rb + 3, jnp.int32)])
        by2 = plsc.load_gather(allc, [jnp.full((LANES,), rb + 4, jnp.int32)])
        bar = plsc.load_gather(allc, [jnp.full((LANES,), rb + 5, jnp.int32)])
        wgi = jnp.full((LANES,), wg.astype(jnp.int32), jnp.int32)
        plsc.subcore_barrier()

        def chunk(j, carry):
            rmax, rgid = carry
            sl = pl.ds(j * LANES, LANES)
            x1 = x1v[sl]
            y1 = y1v[sl]
            x2 = x2v[sl]
            y2 = y2v[sl]
            ar = arv[sl]
            wk = wkv[sl]
            xx1 = jnp.maximum(bx1, x1)
            yy1 = jnp.maximum(by1, y1)
            xx2 = jnp.minimum(bx2, x2)
            yy2 = jnp.minimum(by2, y2)
            iw = jnp.maximum(xx2 - xx1, 0.0)
            ih = jnp.maximum(yy2 - yy1, 0.0)
            inter = iw * ih
            iou = inter / (bar + ar - inter + 1e-9)
            gid = base + j * LANES + lane
            sup = (iou > 0.5) | (gid == wgi)
            nwk = jnp.where(sup, NEG, wk)
            wkv[sl] = nwk
            better = nwk > rmax
            gidf = gid.astype(jnp.float32)
            return (jnp.where(better, nwk, rmax), jnp.where(better, gidf, rgid))

        rmax, rgid = lax.fori_loop(0, CHUNKS, chunk, (rmax0, rgid0))
        return _build_cand(rmax, rgid, lane, x1v, y1v, x2v, y2v, arv, base)

    lax.fori_loop(0, KEEP, step, rec0)

    @pl.when((sub == 0) & (core == 0))
    def _():
        pltpu.sync_copy(outv, out_h)


_nms = functools.partial(
    pl.kernel,
    out_type=jax.ShapeDtypeStruct((KEEP * LANES,), jnp.float32),
    mesh=plsc.VectorSubcoreMesh(core_axis_name="c", subcore_axis_name="s"),
    compiler_params=pltpu.CompilerParams(needs_layout_passes=False),
    scratch_types=[
        pltpu.VMEM((E,), jnp.float32),
        pltpu.VMEM((E,), jnp.float32),
        pltpu.VMEM((E,), jnp.float32),
        pltpu.VMEM((E,), jnp.float32),
        pltpu.VMEM((E,), jnp.float32),
        pltpu.VMEM((E,), jnp.float32),
        pltpu.VMEM((E,), jnp.float32),
        pltpu.VMEM((E,), jnp.float32),
        pltpu.VMEM((E,), jnp.float32),
        pltpu.VMEM((E,), jnp.float32),
        pltpu.VMEM((LANES,), jnp.float32),
        pltpu.VMEM((SUBS * LANES,), jnp.float32),
        pltpu.VMEM((KEEP * LANES,), jnp.float32),
        pltpu.VMEM_SHARED((SUBS * LANES,), jnp.float32),
    ],
)(_nms_body)


def kernel(boxes, scores, box_deltas):
    pad = NPAD - N
    bx1 = jnp.pad(boxes[:, 0], (0, pad))
    by1 = jnp.pad(boxes[:, 1], (0, pad))
    bx2 = jnp.pad(boxes[:, 2], (0, pad))
    by2 = jnp.pad(boxes[:, 3], (0, pad))
    dx = jnp.pad(box_deltas[:, 0], (0, pad))
    dy = jnp.pad(box_deltas[:, 1], (0, pad))
    dw = jnp.pad(box_deltas[:, 2], (0, pad))
    dh = jnp.pad(box_deltas[:, 3], (0, pad))
    sc = jnp.pad(scores, (0, pad), constant_values=PAD_NEG)
    flat = _nms(bx1, by1, bx2, by2, dx, dy, dw, dh, sc)
    o = flat.reshape(KEEP, LANES)
    return jnp.stack([o[:, 1], o[:, 2], o[:, 3], o[:, 4], o[:, 0]], axis=1)


# speedup vs baseline: 12.0717x; 1.1927x over previous
import functools

import jax
import jax.numpy as jnp
from jax import lax
from jax.experimental import pallas as pl
from jax.experimental.pallas import tpu as pltpu
from jax.experimental.pallas import tpu_sc as plsc

N = 20000
NPAD = 20480
SUBS = 16
LANES = 16
E = NPAD // SUBS
CHUNKS = E // LANES
KEEP = 100
IMG = 1024.0
NEG = -1e9
PAD_NEG = -3e9
BIG = 3.0e38


def _build_cand(rmax, rgid, lane, x1v, y1v, x2v, y2v, arv, base):
    m = jnp.max(rmax)
    mv = jnp.full((LANES,), m, jnp.float32)
    g = jnp.min(jnp.where(rmax == mv, rgid, BIG))
    gv = jnp.full((LANES,), g, jnp.float32)
    lidx = g.astype(jnp.int32) - base
    iv = jnp.full((LANES,), lidx, jnp.int32)
    x1w = plsc.load_gather(x1v, [iv])
    y1w = plsc.load_gather(y1v, [iv])
    x2w = plsc.load_gather(x2v, [iv])
    y2w = plsc.load_gather(y2v, [iv])
    arw = plsc.load_gather(arv, [iv])
    rec = jnp.where(
        lane == 0, mv,
        jnp.where(lane == 1, x1w,
                  jnp.where(lane == 2, y1w,
                            jnp.where(lane == 3, x2w,
                                      jnp.where(lane == 4, y2w,
                                                jnp.where(lane == 5, arw, gv))))))
    return rec


def _nms_body(bx1_h, by1_h, bx2_h, by2_h, dx_h, dy_h, dw_h, dh_h, sc_h,
              out_h,
              x1v, y1v, x2v, y2v, dxv, dyv, dwv, dhv, arv, wkv,
              candv, allc, outv, shared):
    sub = lax.axis_index("s")
    core = lax.axis_index("c")
    base = sub * E
    lane = lax.broadcasted_iota(jnp.int32, (LANES,), 0)
    rmax0 = jnp.full((LANES,), -BIG, jnp.float32)
    rgid0 = jnp.zeros((LANES,), jnp.float32)

    pltpu.sync_copy(bx1_h.at[pl.ds(base, E)], x1v)
    pltpu.sync_copy(by1_h.at[pl.ds(base, E)], y1v)
    pltpu.sync_copy(bx2_h.at[pl.ds(base, E)], x2v)
    pltpu.sync_copy(by2_h.at[pl.ds(base, E)], y2v)
    pltpu.sync_copy(dx_h.at[pl.ds(base, E)], dxv)
    pltpu.sync_copy(dy_h.at[pl.ds(base, E)], dyv)
    pltpu.sync_copy(dw_h.at[pl.ds(base, E)], dwv)
    pltpu.sync_copy(dh_h.at[pl.ds(base, E)], dhv)
    pltpu.sync_copy(sc_h.at[pl.ds(base, E)], wkv)

    def decode_chunk(j, carry):
        rmax, rgid = carry
        sl = pl.ds(j * LANES, LANES)
        x1 = x1v[sl]
        y1 = y1v[sl]
        x2 = x2v[sl]
        y2 = y2v[sl]
        w = x2 - x1 + 1.0
        h = y2 - y1 + 1.0
        cx = x1 + 0.5 * w
        cy = y1 + 0.5 * h
        pcx = dxv[sl] * w + cx
        pcy = dyv[sl] * h + cy
        pw = jnp.exp(dwv[sl]) * w
        ph = jnp.exp(dhv[sl]) * h
        nx1 = jnp.clip(pcx - 0.5 * pw, 0.0, IMG)
        ny1 = jnp.clip(pcy - 0.5 * ph, 0.0, IMG)
        nx2 = jnp.clip(pcx + 0.5 * pw, 0.0, IMG)
        ny2 = jnp.clip(pcy + 0.5 * ph, 0.0, IMG)
        ar = jnp.maximum(nx2 - nx1, 0.0) * jnp.maximum(ny2 - ny1, 0.0)
        x1v[sl] = nx1
        y1v[sl] = ny1
        x2v[sl] = nx2
        y2v[sl] = ny2
        arv[sl] = ar
        wk = wkv[sl]
        gidf = (base + j * LANES + lane).astype(jnp.float32)
        better = wk > rmax
        return (jnp.where(better, wk, rmax), jnp.where(better, gidf, rgid))

    rmax, rgid = lax.fori_loop(0, CHUNKS, decode_chunk, (rmax0, rgid0))
    rec0 = _build_cand(rmax, rgid, lane, x1v, y1v, x2v, y2v, arv, base)

    lanef = lane.astype(jnp.float32)
    basef = jnp.float32(E) * sub.astype(jnp.float32)

    def step(t, rec):
        p = lax.bitwise_and(t, 1)
        candv[...] = rec
        pltpu.sync_copy(
            candv, shared.at[pl.ds(p * (SUBS * LANES) + sub * LANES, LANES)])
        plsc.subcore_barrier()
        pltpu.sync_copy(shared.at[pl.ds(p * (SUBS * LANES), SUBS * LANES)],
                        allc)
        vals = plsc.load_gather(allc, [lane * LANES])
        gids = plsc.load_gather(allc, [lane * LANES + 6])
        m = jnp.max(vals)
        tie = vals == jnp.full((LANES,), m, jnp.float32)
        wg = jnp.min(jnp.where(tie, gids, BIG))
        rowc = jnp.where(tie & (gids == jnp.full((LANES,), wg, jnp.float32)),
                         lane, 9999)
        rb = jnp.min(rowc) * LANES
        win = plsc.load_gather(allc, [jnp.full((LANES,), rb, jnp.int32) + lane])

        @pl.when((sub == 0) & (core == 0))
        def _():
            outv[pl.ds(t * LANES, LANES)] = win

        bx1 = plsc.load_gather(allc, [jnp.full((LANES,), rb + 1, jnp.int32)])
        by1 = plsc.load_gather(allc, [jnp.full((LANES,), rb + 2, jnp.int32)])
        bx2 = plsc.load_gather(allc, [jnp.full((LANES,), rb + 3, jnp.int32)])
        by2 = plsc.load_gather(allc, [jnp.full((LANES,), rb + 4, jnp.int32)])
        bar = plsc.load_gather(allc, [jnp.full((LANES,), rb + 5, jnp.int32)])

        wgi = wg.astype(jnp.int32)
        @pl.when((wgi >= base) & (wgi < base + E))
        def _():
            plsc.store_scatter(wkv, [jnp.full((LANES,), wgi - base, jnp.int32)],
                               jnp.full((LANES,), NEG, jnp.float32),
                               mask=lane == 0)

        def chunk(jj, carry):
            rmax, rgid = carry
            for u in range(4):
                j = jj * 4 + u
                sl = pl.ds(j * LANES, LANES)
                x1 = x1v[sl]
                y1 = y1v[sl]
                x2 = x2v[sl]
                y2 = y2v[sl]
                ar = arv[sl]
                wk = wkv[sl]
                xx1 = jnp.maximum(bx1, x1)
                yy1 = jnp.maximum(by1, y1)
                xx2 = jnp.minimum(bx2, x2)
                yy2 = jnp.minimum(by2, y2)
                iw = jnp.maximum(xx2 - xx1, 0.0)
                ih = jnp.maximum(yy2 - yy1, 0.0)
                inter = iw * ih
                iou = inter / (bar + ar - inter + 1e-9)
                nwk = jnp.where(iou > 0.5, NEG, wk)
                wkv[sl] = nwk
                gidf = basef + (j * LANES).astype(jnp.float32) + lanef
                better = nwk > rmax
                rmax = jnp.where(better, nwk, rmax)
                rgid = jnp.where(better, gidf, rgid)
            return (rmax, rgid)

        rmax, rgid = lax.fori_loop(0, CHUNKS // 4, chunk, (rmax0, rgid0))
        return _build_cand(rmax, rgid, lane, x1v, y1v, x2v, y2v, arv, base)

    lax.fori_loop(0, KEEP, step, rec0)

    @pl.when((sub == 0) & (core == 0))
    def _():
        pltpu.sync_copy(outv, out_h)


_nms = functools.partial(
    pl.kernel,
    out_type=jax.ShapeDtypeStruct((KEEP * LANES,), jnp.float32),
    mesh=plsc.VectorSubcoreMesh(core_axis_name="c", subcore_axis_name="s"),
    compiler_params=pltpu.CompilerParams(needs_layout_passes=False),
    scratch_types=[
        pltpu.VMEM((E,), jnp.float32),
        pltpu.VMEM((E,), jnp.float32),
        pltpu.VMEM((E,), jnp.float32),
        pltpu.VMEM((E,), jnp.float32),
        pltpu.VMEM((E,), jnp.float32),
        pltpu.VMEM((E,), jnp.float32),
        pltpu.VMEM((E,), jnp.float32),
        pltpu.VMEM((E,), jnp.float32),
        pltpu.VMEM((E,), jnp.float32),
        pltpu.VMEM((E,), jnp.float32),
        pltpu.VMEM((LANES,), jnp.float32),
        pltpu.VMEM((SUBS * LANES,), jnp.float32),
        pltpu.VMEM((KEEP * LANES,), jnp.float32),
        pltpu.VMEM_SHARED((2 * SUBS * LANES,), jnp.float32),
    ],
)(_nms_body)


def kernel(boxes, scores, box_deltas):
    pad = NPAD - N
    bx1 = jnp.pad(boxes[:, 0], (0, pad))
    by1 = jnp.pad(boxes[:, 1], (0, pad))
    bx2 = jnp.pad(boxes[:, 2], (0, pad))
    by2 = jnp.pad(boxes[:, 3], (0, pad))
    dx = jnp.pad(box_deltas[:, 0], (0, pad))
    dy = jnp.pad(box_deltas[:, 1], (0, pad))
    dw = jnp.pad(box_deltas[:, 2], (0, pad))
    dh = jnp.pad(box_deltas[:, 3], (0, pad))
    sc = jnp.pad(scores, (0, pad), constant_values=PAD_NEG)
    flat = _nms(bx1, by1, bx2, by2, dx, dy, dw, dh, sc)
    o = flat.reshape(KEEP, LANES)
    return jnp.stack([o[:, 1], o[:, 2], o[:, 3], o[:, 4], o[:, 0]], axis=1)
